# baseline (speedup 1.0000x reference)
import functools

import jax
import jax.numpy as jnp
from jax import lax
from jax.experimental import pallas as pl
from jax.experimental.pallas import tpu as pltpu
from jax.experimental.pallas import tpu_sc as plsc

LAT = 64
N_NODES_K = 50000
HALF = 25000
ACC_ROWS = 25008
EP = 819200
CHUNK = 128
NW = 32
CPW = EP // (NW * CHUNK)


def _ln(x, scale, offset):
    mean = jnp.mean(x, axis=-1, keepdims=True)
    xc = x - mean
    var = jnp.mean(xc * xc, axis=-1, keepdims=True)
    return xc * lax.rsqrt(var + 1e-5) * scale + offset


def _dot(a, b):
    return jnp.dot(a, b, preferred_element_type=jnp.float32)


def _enc_body(x_ref, w0, b0, w1, b1, w2, b2, s, o, out_ref):
    h = jnp.maximum(_dot(x_ref[...], w0[...]) + b0[...], 0.0)
    h = jnp.maximum(_dot(h, w1[...]) + b1[...], 0.0)
    h = _dot(h, w2[...]) + b2[...]
    out_ref[...] = _ln(h, s[...], o[...])


def _edge_body(sf_ref, rf_ref, e_ref, ws, wr, we, b0, w1, b1, w2, b2, s, o,
               ne_ref, eo_ref):
    e = e_ref[...]
    h = (_dot(sf_ref[...], ws[...]) + _dot(rf_ref[...], wr[...])
         + _dot(e, we[...]) + b0[...])
    h = jnp.maximum(h, 0.0)
    h = jnp.maximum(_dot(h, w1[...]) + b1[...], 0.0)
    h = _dot(h, w2[...]) + b2[...]
    ne = _ln(h, s[...], o[...])
    ne_ref[...] = ne
    eo_ref[...] = e + ne


def _node_body(n_ref, r_ref, wn, wr, b0, w1, b1, w2, b2, s, o, out_ref):
    n = n_ref[...]
    h = _dot(n, wn[...]) + _dot(r_ref[...], wr[...]) + b0[...]
    h = jnp.maximum(h, 0.0)
    h = jnp.maximum(_dot(h, w1[...]) + b1[...], 0.0)
    h = _dot(h, w2[...]) + b2[...]
    out_ref[...] = n + _ln(h, s[...], o[...])


def _dec_body(x_ref, w0, b0, w1, b1, w2, b2, out_ref):
    h = jnp.maximum(_dot(x_ref[...], w0[...]) + b0[...], 0.0)
    h = jnp.maximum(_dot(h, w1[...]) + b1[...], 0.0)
    out_ref[...] = _dot(h, w2[...]) + b2[...]


def _full_spec(shape):
    return pl.BlockSpec(shape, lambda i: (0,) * len(shape))


def _row_spec(rows, cols):
    return pl.BlockSpec((rows, cols), lambda i: (i, 0))


def _tc_mlp(body, n_rows, block, in_arrays, in_cols, out_cols, n_out=1):
    grid = (n_rows // block,)
    in_specs = [_row_spec(block, c) for c in in_cols]
    in_specs += [_full_spec(a.shape) for a in in_arrays[len(in_cols):]]
    if n_out == 1:
        out_specs = _row_spec(block, out_cols[0])
        out_shape = jax.ShapeDtypeStruct((n_rows, out_cols[0]), jnp.float32)
    else:
        out_specs = [_row_spec(block, c) for c in out_cols]
        out_shape = [jax.ShapeDtypeStruct((n_rows, c), jnp.float32)
                     for c in out_cols]
    return pl.pallas_call(
        body, grid=grid, in_specs=in_specs, out_specs=out_specs,
        out_shape=out_shape)(*in_arrays)


_MESH = plsc.VectorSubcoreMesh(core_axis_name="c", subcore_axis_name="s")


@functools.partial(
    pl.kernel, mesh=_MESH,
    compiler_params=pltpu.CompilerParams(use_tc_tiling_on_sc=False),
    out_type=[jax.ShapeDtypeStruct((EP, LAT), jnp.float32),
              jax.ShapeDtypeStruct((EP, LAT), jnp.float32)],
    scratch_types=[
        pltpu.VMEM((CHUNK,), jnp.int32),
        pltpu.VMEM((CHUNK,), jnp.int32),
        pltpu.VMEM((CHUNK, LAT), jnp.float32),
        pltpu.VMEM((CHUNK, LAT), jnp.float32),
        pltpu.SemaphoreType.DMA,
        pltpu.SemaphoreType.DMA,
    ])
def _sc_gather(nodes_hbm, send_hbm, recv_hbm, sout_hbm, rout_hbm,
               idx_s, idx_r, rows_s, rows_r, sem_s, sem_r):
    c = lax.axis_index("c")
    s = lax.axis_index("s")
    wid = s * 2 + c

    def body(j, carry):
        off = (wid + NW * j) * CHUNK
        pltpu.sync_copy(send_hbm.at[pl.ds(off, CHUNK)], idx_s)
        pltpu.sync_copy(recv_hbm.at[pl.ds(off, CHUNK)], idx_r)
        cp_s = pltpu.async_copy(nodes_hbm.at[idx_s], rows_s, sem_s)
        cp_r = pltpu.async_copy(nodes_hbm.at[idx_r], rows_r, sem_r)
        cp_s.wait()
        cp_r.wait()
        pltpu.sync_copy(rows_s, sout_hbm.at[pl.ds(off, CHUNK)])
        pltpu.sync_copy(rows_r, rout_hbm.at[pl.ds(off, CHUNK)])
        return carry

    lax.fori_loop(0, CPW, body, 0)


SCHUNK = 400


@functools.partial(
    pl.kernel, mesh=_MESH,
    compiler_params=pltpu.CompilerParams(use_tc_tiling_on_sc=False),
    out_type=jax.ShapeDtypeStruct((N_NODES_K, LAT), jnp.float32),
    scratch_types=[
        pltpu.VMEM((SCHUNK,), jnp.int32),
        pltpu.VMEM((SCHUNK, LAT), jnp.float32),
        pltpu.VMEM_SHARED((ACC_ROWS, LAT), jnp.float32),
    ])
def _sc_scatter(ne_hbm, idx_hbm, zeros_hbm, out_hbm, idx_v, rows_v, acc):
    c = lax.axis_index("c")
    s = lax.axis_index("s")
    base = c * HALF
    pltpu.sync_copy(zeros_hbm.at[pl.ds(s * 1563, 1563)],
                    acc.at[pl.ds(s * 1563, 1563)])
    plsc.subcore_barrier()

    @pl.when(s < 2)
    def _adds():
        seg = c * 2 + s

        def body(j, carry):
            off = j * SCHUNK
            pltpu.sync_copy(idx_hbm.at[pl.ds(seg * EP + off, SCHUNK)], idx_v)
            pltpu.sync_copy(ne_hbm.at[pl.ds(off, SCHUNK)], rows_v)
            pltpu.sync_copy(rows_v, acc.at[idx_v], add=True)
            return carry

        lax.fori_loop(0, EP // SCHUNK, body, 0)

    plsc.subcore_barrier()
    pltpu.sync_copy(acc.at[pl.ds(s * 1562, 1562)],
                    out_hbm.at[pl.ds(base + s * 1562, 1562)])

    @pl.when(s == 15)
    def _tail():
        pltpu.sync_copy(acc.at[pl.ds(24992, 8)],
                        out_hbm.at[pl.ds(base + 24992, 8)])


def _b(p, i):
    return p["layers"][i]["b"].reshape(1, -1)


def kernel(vel_hist, vel_mag, bound, rel_disp, rel_dist, params, senders,
           receivers):
    n = vel_hist.shape[0]
    e = senders.shape[0]

    node_feat = jnp.concatenate([vel_hist, vel_mag, bound], axis=-1)
    nf_in = node_feat.shape[1]
    node_feat = jnp.pad(node_feat, ((0, 0), (0, 32 - nf_in)))
    edge_feat = jnp.concatenate([rel_disp, rel_dist], axis=-1)
    edge_feat = jnp.pad(edge_feat, ((0, EP - e), (0, 8 - edge_feat.shape[1])))
    send_p = jnp.pad(senders, (0, EP - e))
    recv_p = jnp.pad(receivers, (0, EP - e), constant_values=N_NODES_K)
    zeros = jnp.zeros((ACC_ROWS, LAT), jnp.float32)

    quarter = HALF // 2

    def localize(base, t):
        q = recv_p - base
        ok = (q >= 0) & (q < quarter)
        return jnp.where(ok, t * quarter + q, HALF + t).astype(jnp.int32)

    recv_loc = jnp.concatenate(
        [localize(c * HALF + t * quarter, t) for c in range(2)
         for t in range(2)])

    pn = params["enc_node"]
    w0 = jnp.pad(pn["layers"][0]["w"], ((0, 32 - nf_in), (0, 0)))
    nodes = _tc_mlp(
        _enc_body, n, 2000,
        [node_feat, w0, _b(pn, 0), pn["layers"][1]["w"], _b(pn, 1),
         pn["layers"][2]["w"], _b(pn, 2), pn["ln_scale"].reshape(1, -1),
         pn["ln_offset"].reshape(1, -1)],
        [32], [LAT])

    pe = params["enc_edge"]
    w0e = jnp.pad(pe["layers"][0]["w"], ((0, 4), (0, 0)))
    edges = _tc_mlp(
        _enc_body, EP, 4096,
        [edge_feat, w0e, _b(pe, 0), pe["layers"][1]["w"], _b(pe, 1),
         pe["layers"][2]["w"], _b(pe, 2), pe["ln_scale"].reshape(1, -1),
         pe["ln_offset"].reshape(1, -1)],
        [8], [LAT])

    for sp in params["proc"]:
        sfeat, rfeat = _sc_gather(nodes, send_p, recv_p)
        pedge = sp["edge"]
        we0 = pedge["layers"][0]["w"]
        new_edges, edges = _tc_mlp(
            _edge_body, EP, 4096,
            [sfeat, rfeat, edges, we0[:LAT], we0[LAT:2 * LAT], we0[2 * LAT:],
             _b(pedge, 0), pedge["layers"][1]["w"], _b(pedge, 1),
             pedge["layers"][2]["w"], _b(pedge, 2),
             pedge["ln_scale"].reshape(1, -1),
             pedge["ln_offset"].reshape(1, -1)],
            [LAT, LAT, LAT], [LAT, LAT], n_out=2)

        received = _sc_scatter(new_edges, recv_loc, zeros)

        pnode = sp["node"]
        wn0 = pnode["layers"][0]["w"]
        nodes = _tc_mlp(
            _node_body, n, 2000,
            [nodes, received, wn0[:LAT], wn0[LAT:], _b(pnode, 0),
             pnode["layers"][1]["w"], _b(pnode, 1), pnode["layers"][2]["w"],
             _b(pnode, 2), pnode["ln_scale"].reshape(1, -1),
             pnode["ln_offset"].reshape(1, -1)],
            [LAT, LAT], [LAT])

    pd = params["dec"]
    w2 = jnp.pad(pd["layers"][2]["w"], ((0, 0), (0, 5)))
    b2 = jnp.pad(pd["layers"][2]["b"], (0, 5)).reshape(1, -1)
    out = _tc_mlp(
        _dec_body, n, 2000,
        [nodes, pd["layers"][0]["w"], _b(pd, 0), pd["layers"][1]["w"],
         _b(pd, 1), w2, b2],
        [LAT], [8])
    return out[:, :3]


# speedup vs baseline: 1.0187x; 1.0187x over previous
import functools

import jax
import jax.numpy as jnp
from jax import lax
from jax.experimental import pallas as pl
from jax.experimental.pallas import tpu as pltpu
from jax.experimental.pallas import tpu_sc as plsc

LAT = 64
N_NODES_K = 50000
HALF = 25000
ACC_ROWS = 25008
EP = 819200
CHUNK = 128
NW = 32
CPW = EP // (NW * CHUNK)


def _ln(x, scale, offset):
    mean = jnp.mean(x, axis=-1, keepdims=True)
    xc = x - mean
    var = jnp.mean(xc * xc, axis=-1, keepdims=True)
    return xc * lax.rsqrt(var + 1e-5) * scale + offset


def _dot(a, b):
    return jnp.dot(a, b, preferred_element_type=jnp.float32)


def _enc_body(x_ref, w0, b0, w1, b1, w2, b2, s, o, out_ref):
    h = jnp.maximum(_dot(x_ref[...], w0[...]) + b0[...], 0.0)
    h = jnp.maximum(_dot(h, w1[...]) + b1[...], 0.0)
    h = _dot(h, w2[...]) + b2[...]
    out_ref[...] = _ln(h, s[...], o[...])


def _edge_body(sf_ref, rf_ref, e_ref, ws, wr, we, b0, w1, b1, w2, b2, s, o,
               ne_ref, eo_ref):
    e = e_ref[...]
    h = (_dot(sf_ref[...], ws[...]) + _dot(rf_ref[...], wr[...])
         + _dot(e, we[...]) + b0[...])
    h = jnp.maximum(h, 0.0)
    h = jnp.maximum(_dot(h, w1[...]) + b1[...], 0.0)
    h = _dot(h, w2[...]) + b2[...]
    ne = _ln(h, s[...], o[...])
    ne_ref[...] = ne
    eo_ref[...] = e + ne


def _node_body(n_ref, r_ref, wn, wr, b0, w1, b1, w2, b2, s, o, out_ref):
    n = n_ref[...]
    h = _dot(n, wn[...]) + _dot(r_ref[...], wr[...]) + b0[...]
    h = jnp.maximum(h, 0.0)
    h = jnp.maximum(_dot(h, w1[...]) + b1[...], 0.0)
    h = _dot(h, w2[...]) + b2[...]
    out_ref[...] = n + _ln(h, s[...], o[...])


def _dec_body(x_ref, w0, b0, w1, b1, w2, b2, out_ref):
    h = jnp.maximum(_dot(x_ref[...], w0[...]) + b0[...], 0.0)
    h = jnp.maximum(_dot(h, w1[...]) + b1[...], 0.0)
    out_ref[...] = _dot(h, w2[...]) + b2[...]


def _full_spec(shape):
    return pl.BlockSpec(shape, lambda i: (0,) * len(shape))


def _row_spec(rows, cols):
    return pl.BlockSpec((rows, cols), lambda i: (i, 0))


def _tc_mlp(body, n_rows, block, in_arrays, in_cols, out_cols, n_out=1):
    grid = (n_rows // block,)
    in_specs = [_row_spec(block, c) for c in in_cols]
    in_specs += [_full_spec(a.shape) for a in in_arrays[len(in_cols):]]
    if n_out == 1:
        out_specs = _row_spec(block, out_cols[0])
        out_shape = jax.ShapeDtypeStruct((n_rows, out_cols[0]), jnp.float32)
    else:
        out_specs = [_row_spec(block, c) for c in out_cols]
        out_shape = [jax.ShapeDtypeStruct((n_rows, c), jnp.float32)
                     for c in out_cols]
    return pl.pallas_call(
        body, grid=grid, in_specs=in_specs, out_specs=out_specs,
        out_shape=out_shape)(*in_arrays)


_MESH = plsc.VectorSubcoreMesh(core_axis_name="c", subcore_axis_name="s")

GCHUNK = 400
GCPW = EP // (NW * GCHUNK)


@functools.partial(
    pl.kernel, mesh=_MESH,
    compiler_params=pltpu.CompilerParams(use_tc_tiling_on_sc=False),
    out_type=[jax.ShapeDtypeStruct((EP, LAT), jnp.float32),
              jax.ShapeDtypeStruct((EP, LAT), jnp.float32)],
    scratch_types=[
        pltpu.VMEM((GCHUNK,), jnp.int32),
        pltpu.VMEM((GCHUNK,), jnp.int32),
        pltpu.VMEM((GCHUNK, LAT), jnp.float32),
        pltpu.VMEM((GCHUNK, LAT), jnp.float32),
        pltpu.SemaphoreType.DMA,
        pltpu.SemaphoreType.DMA,
    ])
def _sc_gather(nodes_hbm, send_hbm, recv_hbm, sout_hbm, rout_hbm,
               idx_s, idx_r, rows_s, rows_r, sem_s, sem_r):
    c = lax.axis_index("c")
    s = lax.axis_index("s")
    wid = s * 2 + c

    def body(j, carry):
        off = (wid + NW * j) * GCHUNK
        pltpu.sync_copy(send_hbm.at[pl.ds(off, GCHUNK)], idx_s)
        pltpu.sync_copy(recv_hbm.at[pl.ds(off, GCHUNK)], idx_r)
        cp_s = pltpu.async_copy(nodes_hbm.at[idx_s], rows_s, sem_s)
        cp_r = pltpu.async_copy(nodes_hbm.at[idx_r], rows_r, sem_r)
        cp_s.wait()
        cp_r.wait()
        pltpu.sync_copy(rows_s, sout_hbm.at[pl.ds(off, GCHUNK)])
        pltpu.sync_copy(rows_r, rout_hbm.at[pl.ds(off, GCHUNK)])
        return carry

    lax.fori_loop(0, GCPW, body, 0)


SCHUNK = 400


@functools.partial(
    pl.kernel, mesh=_MESH,
    compiler_params=pltpu.CompilerParams(use_tc_tiling_on_sc=False),
    out_type=jax.ShapeDtypeStruct((N_NODES_K, LAT), jnp.float32),
    scratch_types=[
        pltpu.VMEM((SCHUNK,), jnp.int32),
        pltpu.VMEM((SCHUNK, LAT), jnp.float32),
        pltpu.VMEM_SHARED((ACC_ROWS, LAT), jnp.float32),
    ])
def _sc_scatter(ne_hbm, idx_hbm, zeros_hbm, out_hbm, idx_v, rows_v, acc):
    c = lax.axis_index("c")
    s = lax.axis_index("s")
    base = c * HALF
    pltpu.sync_copy(zeros_hbm.at[pl.ds(s * 1563, 1563)],
                    acc.at[pl.ds(s * 1563, 1563)])
    plsc.subcore_barrier()

    @pl.when(s == 0)
    def _adds():
        seg = c

        def body(j, carry):
            off = j * SCHUNK
            pltpu.sync_copy(idx_hbm.at[pl.ds(seg * EP + off, SCHUNK)], idx_v)
            pltpu.sync_copy(ne_hbm.at[pl.ds(off, SCHUNK)], rows_v)
            pltpu.sync_copy(rows_v, acc.at[idx_v], add=True)
            return carry

        lax.fori_loop(0, EP // SCHUNK, body, 0)

    plsc.subcore_barrier()
    pltpu.sync_copy(acc.at[pl.ds(s * 1562, 1562)],
                    out_hbm.at[pl.ds(base + s * 1562, 1562)])

    @pl.when(s == 15)
    def _tail():
        pltpu.sync_copy(acc.at[pl.ds(24992, 8)],
                        out_hbm.at[pl.ds(base + 24992, 8)])


def _b(p, i):
    return p["layers"][i]["b"].reshape(1, -1)


def kernel(vel_hist, vel_mag, bound, rel_disp, rel_dist, params, senders,
           receivers):
    n = vel_hist.shape[0]
    e = senders.shape[0]

    node_feat = jnp.concatenate([vel_hist, vel_mag, bound], axis=-1)
    nf_in = node_feat.shape[1]
    node_feat = jnp.pad(node_feat, ((0, 0), (0, 32 - nf_in)))
    edge_feat = jnp.concatenate([rel_disp, rel_dist], axis=-1)
    edge_feat = jnp.pad(edge_feat, ((0, EP - e), (0, 8 - edge_feat.shape[1])))
    send_p = jnp.pad(senders, (0, EP - e))
    recv_p = jnp.pad(receivers, (0, EP - e), constant_values=N_NODES_K)
    zeros = jnp.zeros((ACC_ROWS, LAT), jnp.float32)

    def localize(base):
        loc = recv_p - base
        ok = (loc >= 0) & (loc < HALF)
        return jnp.where(ok, loc, HALF).astype(jnp.int32)

    recv_loc = jnp.concatenate([localize(0), localize(HALF)])

    pn = params["enc_node"]
    w0 = jnp.pad(pn["layers"][0]["w"], ((0, 32 - nf_in), (0, 0)))
    nodes = _tc_mlp(
        _enc_body, n, 2000,
        [node_feat, w0, _b(pn, 0), pn["layers"][1]["w"], _b(pn, 1),
         pn["layers"][2]["w"], _b(pn, 2), pn["ln_scale"].reshape(1, -1),
         pn["ln_offset"].reshape(1, -1)],
        [32], [LAT])

    pe = params["enc_edge"]
    w0e = jnp.pad(pe["layers"][0]["w"], ((0, 4), (0, 0)))
    edges = _tc_mlp(
        _enc_body, EP, 4096,
        [edge_feat, w0e, _b(pe, 0), pe["layers"][1]["w"], _b(pe, 1),
         pe["layers"][2]["w"], _b(pe, 2), pe["ln_scale"].reshape(1, -1),
         pe["ln_offset"].reshape(1, -1)],
        [8], [LAT])

    for sp in params["proc"]:
        sfeat, rfeat = _sc_gather(nodes, send_p, recv_p)
        pedge = sp["edge"]
        we0 = pedge["layers"][0]["w"]
        new_edges, edges = _tc_mlp(
            _edge_body, EP, 4096,
            [sfeat, rfeat, edges, we0[:LAT], we0[LAT:2 * LAT], we0[2 * LAT:],
             _b(pedge, 0), pedge["layers"][1]["w"], _b(pedge, 1),
             pedge["layers"][2]["w"], _b(pedge, 2),
             pedge["ln_scale"].reshape(1, -1),
             pedge["ln_offset"].reshape(1, -1)],
            [LAT, LAT, LAT], [LAT, LAT], n_out=2)

        received = _sc_scatter(new_edges, recv_loc, zeros)

        pnode = sp["node"]
        wn0 = pnode["layers"][0]["w"]
        nodes = _tc_mlp(
            _node_body, n, 2000,
            [nodes, received, wn0[:LAT], wn0[LAT:], _b(pnode, 0),
             pnode["layers"][1]["w"], _b(pnode, 1), pnode["layers"][2]["w"],
             _b(pnode, 2), pnode["ln_scale"].reshape(1, -1),
             pnode["ln_offset"].reshape(1, -1)],
            [LAT, LAT], [LAT])

    pd = params["dec"]
    w2 = jnp.pad(pd["layers"][2]["w"], ((0, 0), (0, 5)))
    b2 = jnp.pad(pd["layers"][2]["b"], (0, 5)).reshape(1, -1)
    out = _tc_mlp(
        _dec_body, n, 2000,
        [nodes, pd["layers"][0]["w"], _b(pd, 0), pd["layers"][1]["w"],
         _b(pd, 1), w2, b2],
        [LAT], [8])
    return out[:, :3]
